# baseline (speedup 1.0000x reference)
import functools

import jax
import jax.numpy as jnp
from jax import lax
from jax.experimental import pallas as pl
from jax.experimental.pallas import tpu as pltpu
from jax.experimental.pallas import tpu_sc as plsc

N = 10000
E = 320000
D = 128
H = 8
DH = 16

NC, NS = 2, 16
NW = NC * NS
NP = 10240
RPT = NP // NS
EPT = E // NW
C = 80
NCH = EPT // C

BN = 1000
NB = N // BN

_f32 = jnp.float32


def _dense_pre_body(x_ref, w_ref, alf_ref, arf_ref, sel_ref,
                    feat_ref, el_ref, er_ref, mel_ref):
    i = pl.program_id(0)
    feat = jnp.dot(x_ref[...], w_ref[...], preferred_element_type=_f32)
    feat_ref[...] = feat
    el = jnp.dot(feat * alf_ref[...], sel_ref[...], preferred_element_type=_f32)
    er = jnp.dot(feat * arf_ref[...], sel_ref[...], preferred_element_type=_f32)
    el_ref[...] = el
    er_ref[...] = er
    bm = jnp.max(el, axis=0, keepdims=True)

    @pl.when(i == 0)
    def _():
        mel_ref[...] = bm

    @pl.when(i > 0)
    def _():
        mel_ref[...] = jnp.maximum(mel_ref[...], bm)


def _dense_pre(x, w, alf, arf, sel):
    return pl.pallas_call(
        _dense_pre_body,
        grid=(NB,),
        in_specs=[
            pl.BlockSpec((BN, D), lambda i: (i, 0)),
            pl.BlockSpec((D, D), lambda i: (0, 0)),
            pl.BlockSpec((1, D), lambda i: (0, 0)),
            pl.BlockSpec((1, D), lambda i: (0, 0)),
            pl.BlockSpec((D, H), lambda i: (0, 0)),
        ],
        out_specs=[
            pl.BlockSpec((BN, D), lambda i: (i, 0)),
            pl.BlockSpec((BN, H), lambda i: (i, 0)),
            pl.BlockSpec((BN, H), lambda i: (i, 0)),
            pl.BlockSpec((1, H), lambda i: (0, 0)),
        ],
        out_shape=[
            jax.ShapeDtypeStruct((N, D), _f32),
            jax.ShapeDtypeStruct((N, H), _f32),
            jax.ShapeDtypeStruct((N, H), _f32),
            jax.ShapeDtypeStruct((1, H), _f32),
        ],
    )(x, w, alf, arf, sel)


def _dstt_body(er_ref, mel_ref, dt_ref):
    er = er_ref[...]
    t = er + mel_ref[...]
    b = jnp.maximum(t, 0.2 * t)
    dt_ref[...] = jnp.concatenate([er, b], axis=1)


def _mk_dstt(er, mel):
    return pl.pallas_call(
        _dstt_body,
        grid=(NB,),
        in_specs=[
            pl.BlockSpec((BN, H), lambda i: (i, 0)),
            pl.BlockSpec((1, H), lambda i: (0, 0)),
        ],
        out_specs=pl.BlockSpec((BN, 2 * H), lambda i: (i, 0)),
        out_shape=jax.ShapeDtypeStruct((N, 2 * H), _f32),
    )(er, mel)


def _dense_mid_body(acc_ref, den_ref, brow_ref, w_ref, alf_ref, arf_ref,
                    sel_ref, selt_ref,
                    feat_ref, el_ref, er_ref, mel_ref):
    i = pl.program_id(0)
    acc = acc_ref[0] + acc_ref[1]
    den = den_ref[0] + den_ref[1]
    deninv = 1.0 / (den + 1e-9)
    dexp = jnp.dot(deninv, selt_ref[...], preferred_element_type=_f32)
    h = acc * dexp + brow_ref[...]
    h = jnp.where(h > 0, h, jnp.exp(jnp.minimum(h, 0.0)) - 1.0)
    feat = jnp.dot(h, w_ref[...], preferred_element_type=_f32)
    feat_ref[...] = feat
    el = jnp.dot(feat * alf_ref[...], sel_ref[...], preferred_element_type=_f32)
    er = jnp.dot(feat * arf_ref[...], sel_ref[...], preferred_element_type=_f32)
    el_ref[...] = el
    er_ref[...] = er
    bm = jnp.max(el, axis=0, keepdims=True)

    @pl.when(i == 0)
    def _():
        mel_ref[...] = bm

    @pl.when(i > 0)
    def _():
        mel_ref[...] = jnp.maximum(mel_ref[...], bm)


def _dense_mid(acc, den, brow, w, alf, arf, sel, selt):
    return pl.pallas_call(
        _dense_mid_body,
        grid=(NB,),
        in_specs=[
            pl.BlockSpec((NC, BN, D), lambda i: (0, i, 0)),
            pl.BlockSpec((NC, BN, H), lambda i: (0, i, 0)),
            pl.BlockSpec((1, D), lambda i: (0, 0)),
            pl.BlockSpec((D, D), lambda i: (0, 0)),
            pl.BlockSpec((1, D), lambda i: (0, 0)),
            pl.BlockSpec((1, D), lambda i: (0, 0)),
            pl.BlockSpec((D, H), lambda i: (0, 0)),
            pl.BlockSpec((H, D), lambda i: (0, 0)),
        ],
        out_specs=[
            pl.BlockSpec((BN, D), lambda i: (i, 0)),
            pl.BlockSpec((BN, H), lambda i: (i, 0)),
            pl.BlockSpec((BN, H), lambda i: (i, 0)),
            pl.BlockSpec((1, H), lambda i: (0, 0)),
        ],
        out_shape=[
            jax.ShapeDtypeStruct((N, D), _f32),
            jax.ShapeDtypeStruct((N, H), _f32),
            jax.ShapeDtypeStruct((N, H), _f32),
            jax.ShapeDtypeStruct((1, H), _f32),
        ],
    )(acc, den, brow, w, alf, arf, sel, selt)


def _dense_fin_body(acc_ref, den_ref, brow_ref, selt_ref, out_ref):
    acc = acc_ref[0] + acc_ref[1]
    den = den_ref[0] + den_ref[1]
    deninv = 1.0 / (den + 1e-9)
    dexp = jnp.dot(deninv, selt_ref[...], preferred_element_type=_f32)
    h = acc * dexp + brow_ref[...]
    out_ref[...] = jnp.where(h > 0, h, jnp.exp(jnp.minimum(h, 0.0)) - 1.0)


def _dense_fin(acc, den, brow, selt):
    return pl.pallas_call(
        _dense_fin_body,
        grid=(NB,),
        in_specs=[
            pl.BlockSpec((NC, BN, D), lambda i: (0, i, 0)),
            pl.BlockSpec((NC, BN, H), lambda i: (0, i, 0)),
            pl.BlockSpec((1, D), lambda i: (0, 0)),
            pl.BlockSpec((H, D), lambda i: (0, 0)),
        ],
        out_specs=pl.BlockSpec((BN, D), lambda i: (i, 0)),
        out_shape=jax.ShapeDtypeStruct((N, D), _f32),
    )(acc, den, brow, selt)


def _sc_edge_body(src_h, dst_h, ew_h, el_h, dt_h, feat_h, z128_h, z8_h,
                  acc_o, den_o,
                  sidx0, didx0, ewv0, elg0, dtg0, featg0, pch0,
                  sidx1, didx1, ewv1, elg1, dtg1, featg1, pch1,
                  acc_sh, den_sh, semi0, semg0, sems0, semi1, semg1, sems1):
    cid = lax.axis_index("c")
    sid = lax.axis_index("s")
    wid = cid * NS + sid
    row0 = sid * RPT

    pltpu.sync_copy(z128_h.at[pl.ds(row0, RPT)], acc_sh.at[pl.ds(row0, RPT)])
    pltpu.sync_copy(z8_h.at[pl.ds(row0, RPT)], den_sh.at[pl.ds(row0, RPT)])
    plsc.subcore_barrier()

    ebase = wid * EPT
    B0 = (sidx0, didx0, ewv0, elg0, dtg0, featg0, pch0, semi0, semg0, sems0)
    B1 = (sidx1, didx1, ewv1, elg1, dtg1, featg1, pch1, semi1, semg1, sems1)

    def issue_idx(c, b):
        base = ebase + c * C
        pltpu.async_copy(src_h.at[pl.ds(base, C)], b[0], b[7])
        pltpu.async_copy(dst_h.at[pl.ds(base, C)], b[1], b[7])
        pltpu.async_copy(ew_h.at[pl.ds(base, C)], b[2], b[7])

    def wait_idx(b):
        pltpu.make_async_copy(src_h.at[pl.ds(0, C)], b[0], b[7]).wait()
        pltpu.make_async_copy(dst_h.at[pl.ds(0, C)], b[1], b[7]).wait()
        pltpu.make_async_copy(ew_h.at[pl.ds(0, C)], b[2], b[7]).wait()

    def issue_gath(b):
        pltpu.async_copy(el_h.at[b[0]], b[3], b[8])
        pltpu.async_copy(dt_h.at[b[1]], b[4], b[8])
        pltpu.async_copy(feat_h.at[b[0]], b[5], b[8])

    def wait_gath(b):
        pltpu.make_async_copy(el_h.at[b[0]], b[3], b[8]).wait()
        pltpu.make_async_copy(dt_h.at[b[1]], b[4], b[8]).wait()
        pltpu.make_async_copy(feat_h.at[b[0]], b[5], b[8]).wait()

    def issue_scat(b):
        pltpu.async_copy(b[6], den_sh.at[b[1]], b[9], add=True)
        pltpu.async_copy(b[5], acc_sh.at[b[1]], b[9], add=True)

    def wait_scat(b):
        pltpu.make_async_copy(b[6], den_sh.at[b[1]], b[9]).wait()
        pltpu.make_async_copy(b[5], acc_sh.at[b[1]], b[9]).wait()

    def compute(b):
        elg, dtg, featg, pch, ewv = b[3], b[4], b[5], b[6], b[2]

        @plsc.parallel_loop(0, C // 2, 1, unroll=4)
        def pair_body(i):
            it = lax.iota(jnp.int32, 16)
            half = lax.shift_right_logical(it, 3)
            col8 = jnp.bitwise_and(it, 7)
            e0 = 2 * i
            row2 = e0 + half
            el2 = plsc.load_gather(elg, [row2, col8])
            er2 = plsc.load_gather(dtg, [row2, col8])
            b2 = plsc.load_gather(dtg, [row2, col8 + 8])
            ew2 = plsc.load_gather(ewv, [row2])
            s = el2 + er2
            lr = jnp.maximum(s, 0.2 * s)
            p = jnp.exp(lr - b2)
            q = p * ew2
            plsc.store_scatter(pch, [row2, col8], p)
            for sub in range(2):
                e = e0 + sub
                for hh in range(H):
                    qs = jnp.take(q, jnp.full((16,), sub * H + hh, jnp.int32),
                                  mode="fill")
                    featg[e, pl.ds(hh * DH, DH)] = featg[e, pl.ds(hh * DH, DH)] * qs

    issue_idx(0, B0)
    wait_idx(B0)
    issue_gath(B0)
    NJ = NCH // 2

    def pipe_body(j, carry):
        c1 = 2 * j + 1
        issue_idx(c1, B1)
        wait_gath(B0)
        wait_idx(B1)
        issue_gath(B1)
        compute(B0)
        issue_scat(B0)
        wait_gath(B1)
        compute(B1)
        wait_scat(B0)

        @pl.when(j < NJ - 1)
        def _():
            issue_idx(c1 + 1, B0)

        pltpu.sync_copy(B1[6], den_sh.at[B1[1]], add=True)
        pltpu.sync_copy(B1[5], acc_sh.at[B1[1]], add=True)

        @pl.when(j < NJ - 1)
        def _():
            wait_idx(B0)
            issue_gath(B0)

        return carry

    lax.fori_loop(0, NJ, pipe_body, 0)

    if NCH % 2 == 1:
        issue_idx(NCH - 1, B0)
        wait_idx(B0)
        issue_gath(B0)
        wait_gath(B0)
        compute(B0)
        pltpu.sync_copy(B0[6], den_sh.at[B0[1]], add=True)
        pltpu.sync_copy(B0[5], acc_sh.at[B0[1]], add=True)

    plsc.subcore_barrier()
    pltpu.sync_copy(acc_sh.at[pl.ds(row0, RPT)], acc_o.at[cid, pl.ds(row0, RPT)])
    pltpu.sync_copy(den_sh.at[pl.ds(row0, RPT)], den_o.at[cid, pl.ds(row0, RPT)])


@functools.cache
def _get_sc_edge_pass():
  return functools.partial(
    pl.kernel,
    out_type=(jax.ShapeDtypeStruct((NC, NP, D), _f32),
              jax.ShapeDtypeStruct((NC, NP, H), _f32)),
    mesh=plsc.VectorSubcoreMesh(core_axis_name="c", subcore_axis_name="s",
                                num_cores=NC, num_subcores=NS),
    compiler_params=pltpu.CompilerParams(use_tc_tiling_on_sc=False, needs_layout_passes=False),
    scratch_types=(
        [pltpu.VMEM((C,), jnp.int32),
         pltpu.VMEM((C,), jnp.int32),
         pltpu.VMEM((C,), _f32),
         pltpu.VMEM((C, H), _f32),
         pltpu.VMEM((C, 2 * H), _f32),
         pltpu.VMEM((C, D), _f32),
         pltpu.VMEM((C, H), _f32)] * 2
        + [pltpu.VMEM_SHARED((NP, D), _f32),
           pltpu.VMEM_SHARED((NP, H), _f32)]
        + [pltpu.SemaphoreType.DMA] * 6
    ),
  )(_sc_edge_body)


def kernel(in_feat, edge_index, edge_weights, W1, attn_l1, attn_r1, b1,
           W2, attn_l2, attn_r2, b2):
    src = edge_index[0]
    dst = edge_index[1]
    sel = jnp.repeat(jnp.eye(H, dtype=_f32), DH, axis=0)
    selt = sel.T
    alf1 = attn_l1.reshape(1, D)
    arf1 = attn_r1.reshape(1, D)
    alf2 = attn_l2.reshape(1, D)
    arf2 = attn_r2.reshape(1, D)
    z128 = jnp.zeros((NP, D), _f32)
    z8 = jnp.zeros((NP, H), _f32)

    feat1, el1, er1, mel1 = _dense_pre(in_feat, W1, alf1, arf1, sel)
    dt1 = _mk_dstt(er1, mel1)
    sc_pass = _get_sc_edge_pass()
    acc1, den1 = sc_pass(src, dst, edge_weights, el1, dt1, feat1, z128, z8)
    feat2, el2, er2, mel2 = _dense_mid(acc1, den1, b1.reshape(1, D), W2,
                                       alf2, arf2, sel, selt)
    dt2 = _mk_dstt(er2, mel2)
    acc2, den2 = sc_pass(src, dst, edge_weights, el2, dt2, feat2, z128, z8)
    out = _dense_fin(acc2, den2, b2.reshape(1, D), selt)
    return out


# speedup vs baseline: 1.0917x; 1.0917x over previous
import functools

import jax
import jax.numpy as jnp
from jax import lax
from jax.experimental import pallas as pl
from jax.experimental.pallas import tpu as pltpu
from jax.experimental.pallas import tpu_sc as plsc

N = 10000
E = 320000
D = 128
H = 8
DH = 16
DX = D + H

NC, NS = 2, 16
NW = NC * NS
NP = 10240
RPT = NP // NS
EPT = E // NW
C = 80
NCH = EPT // C

BN = 1000
NB = N // BN

_f32 = jnp.float32


def _dense_pre_body(x_ref, w_ref, alf_ref, arf_ref, sel_ref,
                    featx_ref, er_ref, mel_ref):
    i = pl.program_id(0)
    feat = jnp.dot(x_ref[...], w_ref[...], preferred_element_type=_f32)
    el = jnp.dot(feat * alf_ref[...], sel_ref[...], preferred_element_type=_f32)
    er = jnp.dot(feat * arf_ref[...], sel_ref[...], preferred_element_type=_f32)
    featx_ref[...] = jnp.concatenate([feat, el], axis=1)
    er_ref[...] = er
    bm = jnp.max(el, axis=0, keepdims=True)

    @pl.when(i == 0)
    def _():
        mel_ref[...] = bm

    @pl.when(i > 0)
    def _():
        mel_ref[...] = jnp.maximum(mel_ref[...], bm)


def _dense_pre(x, w, alf, arf, sel):
    return pl.pallas_call(
        _dense_pre_body,
        grid=(NB,),
        in_specs=[
            pl.BlockSpec((BN, D), lambda i: (i, 0)),
            pl.BlockSpec((D, D), lambda i: (0, 0)),
            pl.BlockSpec((1, D), lambda i: (0, 0)),
            pl.BlockSpec((1, D), lambda i: (0, 0)),
            pl.BlockSpec((D, H), lambda i: (0, 0)),
        ],
        out_specs=[
            pl.BlockSpec((BN, DX), lambda i: (i, 0)),
            pl.BlockSpec((BN, H), lambda i: (i, 0)),
            pl.BlockSpec((1, H), lambda i: (0, 0)),
        ],
        out_shape=[
            jax.ShapeDtypeStruct((N, DX), _f32),
            jax.ShapeDtypeStruct((N, H), _f32),
            jax.ShapeDtypeStruct((1, H), _f32),
        ],
    )(x, w, alf, arf, sel)


def _dstt_body(er_ref, mel_ref, dt_ref):
    er = er_ref[...]
    t = er + mel_ref[...]
    b = jnp.maximum(t, 0.2 * t)
    dt_ref[...] = jnp.concatenate([er, b], axis=1)


def _mk_dstt(er, mel):
    return pl.pallas_call(
        _dstt_body,
        grid=(NB,),
        in_specs=[
            pl.BlockSpec((BN, H), lambda i: (i, 0)),
            pl.BlockSpec((1, H), lambda i: (0, 0)),
        ],
        out_specs=pl.BlockSpec((BN, 2 * H), lambda i: (i, 0)),
        out_shape=jax.ShapeDtypeStruct((N, 2 * H), _f32),
    )(er, mel)


def _dense_mid_body(acc_ref, brow_ref, w_ref, alf_ref, arf_ref,
                    sel_ref, selt_ref,
                    featx_ref, er_ref, mel_ref):
    i = pl.program_id(0)
    accx = acc_ref[0] + acc_ref[1]
    acc = accx[:, :D]
    den = accx[:, D:]
    deninv = 1.0 / (den + 1e-9)
    dexp = jnp.dot(deninv, selt_ref[...], preferred_element_type=_f32)
    h = acc * dexp + brow_ref[...]
    h = jnp.where(h > 0, h, jnp.exp(jnp.minimum(h, 0.0)) - 1.0)
    feat = jnp.dot(h, w_ref[...], preferred_element_type=_f32)
    el = jnp.dot(feat * alf_ref[...], sel_ref[...], preferred_element_type=_f32)
    er = jnp.dot(feat * arf_ref[...], sel_ref[...], preferred_element_type=_f32)
    featx_ref[...] = jnp.concatenate([feat, el], axis=1)
    er_ref[...] = er
    bm = jnp.max(el, axis=0, keepdims=True)

    @pl.when(i == 0)
    def _():
        mel_ref[...] = bm

    @pl.when(i > 0)
    def _():
        mel_ref[...] = jnp.maximum(mel_ref[...], bm)


def _dense_mid(acc, brow, w, alf, arf, sel, selt):
    return pl.pallas_call(
        _dense_mid_body,
        grid=(NB,),
        in_specs=[
            pl.BlockSpec((NC, BN, DX), lambda i: (0, i, 0)),
            pl.BlockSpec((1, D), lambda i: (0, 0)),
            pl.BlockSpec((D, D), lambda i: (0, 0)),
            pl.BlockSpec((1, D), lambda i: (0, 0)),
            pl.BlockSpec((1, D), lambda i: (0, 0)),
            pl.BlockSpec((D, H), lambda i: (0, 0)),
            pl.BlockSpec((H, D), lambda i: (0, 0)),
        ],
        out_specs=[
            pl.BlockSpec((BN, DX), lambda i: (i, 0)),
            pl.BlockSpec((BN, H), lambda i: (i, 0)),
            pl.BlockSpec((1, H), lambda i: (0, 0)),
        ],
        out_shape=[
            jax.ShapeDtypeStruct((N, DX), _f32),
            jax.ShapeDtypeStruct((N, H), _f32),
            jax.ShapeDtypeStruct((1, H), _f32),
        ],
    )(acc, brow, w, alf, arf, sel, selt)


def _dense_fin_body(acc_ref, brow_ref, selt_ref, out_ref):
    accx = acc_ref[0] + acc_ref[1]
    acc = accx[:, :D]
    den = accx[:, D:]
    deninv = 1.0 / (den + 1e-9)
    dexp = jnp.dot(deninv, selt_ref[...], preferred_element_type=_f32)
    h = acc * dexp + brow_ref[...]
    out_ref[...] = jnp.where(h > 0, h, jnp.exp(jnp.minimum(h, 0.0)) - 1.0)


def _dense_fin(acc, brow, selt):
    return pl.pallas_call(
        _dense_fin_body,
        grid=(NB,),
        in_specs=[
            pl.BlockSpec((NC, BN, DX), lambda i: (0, i, 0)),
            pl.BlockSpec((1, D), lambda i: (0, 0)),
            pl.BlockSpec((H, D), lambda i: (0, 0)),
        ],
        out_specs=pl.BlockSpec((BN, D), lambda i: (i, 0)),
        out_shape=jax.ShapeDtypeStruct((N, D), _f32),
    )(acc, brow, selt)


def _sc_edge_body(ed_h, dt_h, featx_h, zx_h,
                  acc_o,
                  ebuf0, dtg0, featg0,
                  ebuf1, dtg1, featg1,
                  acc_sh, semi0, semg0, sems0, semi1, semg1, sems1):
    cid = lax.axis_index("c")
    sid = lax.axis_index("s")
    wid = cid * NS + sid
    row0 = sid * RPT

    pltpu.sync_copy(zx_h.at[pl.ds(row0, RPT)], acc_sh.at[pl.ds(row0, RPT)])
    plsc.subcore_barrier()

    cbase = wid * NCH
    B0 = (ebuf0, dtg0, featg0, semi0, semg0, sems0)
    B1 = (ebuf1, dtg1, featg1, semi1, semg1, sems1)

    def issue_idx(c, b):
        pltpu.async_copy(ed_h.at[cbase + c], b[0], b[3])

    def wait_idx(b):
        pltpu.make_async_copy(ed_h.at[0], b[0], b[3]).wait()

    def issue_gath(b):
        pltpu.async_copy(dt_h.at[b[0].at[1]], b[1], b[4])
        pltpu.async_copy(featx_h.at[b[0].at[0]], b[2], b[4])

    def wait_gath(b):
        pltpu.make_async_copy(dt_h.at[b[0].at[1]], b[1], b[4]).wait()
        pltpu.make_async_copy(featx_h.at[b[0].at[0]], b[2], b[4]).wait()

    def issue_scat(b):
        pltpu.async_copy(b[2], acc_sh.at[b[0].at[1]], b[5], add=True)

    def wait_scat(b):
        pltpu.make_async_copy(b[2], acc_sh.at[b[0].at[1]], b[5]).wait()

    def compute(b):
        ebuf, dtg, featg = b[0], b[1], b[2]

        @plsc.parallel_loop(0, C // 2, 1, unroll=2)
        def pair_body(i):
            it = lax.iota(jnp.int32, 16)
            half = lax.shift_right_logical(it, 3)
            col8 = jnp.bitwise_and(it, 7)
            e0 = 2 * i
            row2 = e0 + half
            el2 = plsc.load_gather(featg, [row2, col8 + D])
            er2 = plsc.load_gather(dtg, [row2, col8])
            b2 = plsc.load_gather(dtg, [row2, col8 + 8])
            ewi = plsc.load_gather(ebuf, [jnp.full((16,), 2, jnp.int32), row2])
            ew2 = plsc.bitcast(ewi, _f32)
            s = el2 + er2
            lr = jnp.maximum(s, 0.2 * s)
            p = jnp.exp(lr - b2)
            q = p * ew2
            plsc.store_scatter(featg, [row2, col8 + D], p)
            for sub in range(2):
                e = e0 + sub
                for hh in range(H):
                    qs = jnp.take(q, jnp.full((16,), sub * H + hh, jnp.int32),
                                  mode="fill")
                    featg[e, pl.ds(hh * DH, DH)] = featg[e, pl.ds(hh * DH, DH)] * qs

    issue_idx(0, B0)
    wait_idx(B0)
    issue_gath(B0)
    NJ = NCH // 2

    def pipe_body(j, carry):
        c1 = 2 * j + 1
        issue_idx(c1, B1)
        wait_gath(B0)
        wait_idx(B1)
        issue_gath(B1)
        compute(B0)
        issue_scat(B0)
        wait_gath(B1)
        compute(B1)
        wait_scat(B0)

        @pl.when(j < NJ - 1)
        def _():
            issue_idx(c1 + 1, B0)

        pltpu.sync_copy(B1[2], acc_sh.at[B1[0].at[1]], add=True)

        @pl.when(j < NJ - 1)
        def _():
            wait_idx(B0)
            issue_gath(B0)

        return carry

    lax.fori_loop(0, NJ, pipe_body, 0)

    if NCH % 2 == 1:
        issue_idx(NCH - 1, B0)
        wait_idx(B0)
        issue_gath(B0)
        wait_gath(B0)
        compute(B0)
        pltpu.sync_copy(B0[2], acc_sh.at[B0[0].at[1]], add=True)

    plsc.subcore_barrier()
    pltpu.sync_copy(acc_sh.at[pl.ds(row0, RPT)], acc_o.at[cid, pl.ds(row0, RPT)])


@functools.cache
def _get_sc_edge_pass():
  return functools.partial(
    pl.kernel,
    out_type=jax.ShapeDtypeStruct((NC, NP, DX), _f32),
    mesh=plsc.VectorSubcoreMesh(core_axis_name="c", subcore_axis_name="s",
                                num_cores=NC, num_subcores=NS),
    compiler_params=pltpu.CompilerParams(use_tc_tiling_on_sc=False, needs_layout_passes=False),
    scratch_types=(
        [pltpu.VMEM((3, C), jnp.int32),
         pltpu.VMEM((C, 2 * H), _f32),
         pltpu.VMEM((C, DX), _f32)] * 2
        + [pltpu.VMEM_SHARED((NP, DX), _f32)]
        + [pltpu.SemaphoreType.DMA] * 6
    ),
  )(_sc_edge_body)


def kernel(in_feat, edge_index, edge_weights, W1, attn_l1, attn_r1, b1,
           W2, attn_l2, attn_r2, b2):
    src = edge_index[0]
    dst = edge_index[1]
    ewbits = lax.bitcast_convert_type(edge_weights, jnp.int32)
    edata = jnp.stack([src.reshape(E // C, C), dst.reshape(E // C, C),
                       ewbits.reshape(E // C, C)], axis=1)
    sel = jnp.repeat(jnp.eye(H, dtype=_f32), DH, axis=0)
    selt = sel.T
    alf1 = attn_l1.reshape(1, D)
    arf1 = attn_r1.reshape(1, D)
    alf2 = attn_l2.reshape(1, D)
    arf2 = attn_r2.reshape(1, D)
    zx = jnp.zeros((NP, DX), _f32)

    featx1, er1, mel1 = _dense_pre(in_feat, W1, alf1, arf1, sel)
    dt1 = _mk_dstt(er1, mel1)
    sc_pass = _get_sc_edge_pass()
    acc1 = sc_pass(edata, dt1, featx1, zx)
    featx2, er2, mel2 = _dense_mid(acc1, b1.reshape(1, D), W2,
                                   alf2, arf2, sel, selt)
    dt2 = _mk_dstt(er2, mel2)
    acc2 = sc_pass(edata, dt2, featx2, zx)
    out = _dense_fin(acc2, b2.reshape(1, D), selt)
    return out


# speedup vs baseline: 1.1144x; 1.0208x over previous
import functools

import jax
import jax.numpy as jnp
from jax import lax
from jax.experimental import pallas as pl
from jax.experimental.pallas import tpu as pltpu
from jax.experimental.pallas import tpu_sc as plsc

N = 10000
E = 320000
D = 128
H = 8
DH = 16
DX = D + 2 * H

NC, NS = 2, 16
NW = NC * NS
NP = 10240
RPT = NP // NS
EPT = E // NW
C = 80
NCH = EPT // C

BN = 1000
NB = N // BN

_f32 = jnp.float32


def _dense_pre_body(x_ref, w_ref, alf_ref, arf_ref, sel_ref,
                    featx_ref, er_ref, mel_ref):
    i = pl.program_id(0)
    feat = jnp.dot(x_ref[...], w_ref[...], preferred_element_type=_f32)
    el = jnp.dot(feat * alf_ref[...], sel_ref[...], preferred_element_type=_f32)
    er = jnp.dot(feat * arf_ref[...], sel_ref[...], preferred_element_type=_f32)
    featx_ref[...] = jnp.concatenate([feat, el, jnp.zeros((BN, H), _f32)], axis=1)
    er_ref[...] = er
    bm = jnp.max(el, axis=0, keepdims=True)

    @pl.when(i == 0)
    def _():
        mel_ref[...] = bm

    @pl.when(i > 0)
    def _():
        mel_ref[...] = jnp.maximum(mel_ref[...], bm)


def _dense_pre(x, w, alf, arf, sel):
    return pl.pallas_call(
        _dense_pre_body,
        grid=(NB,),
        in_specs=[
            pl.BlockSpec((BN, D), lambda i: (i, 0)),
            pl.BlockSpec((D, D), lambda i: (0, 0)),
            pl.BlockSpec((1, D), lambda i: (0, 0)),
            pl.BlockSpec((1, D), lambda i: (0, 0)),
            pl.BlockSpec((D, H), lambda i: (0, 0)),
        ],
        out_specs=[
            pl.BlockSpec((BN, DX), lambda i: (i, 0)),
            pl.BlockSpec((BN, H), lambda i: (i, 0)),
            pl.BlockSpec((1, H), lambda i: (0, 0)),
        ],
        out_shape=[
            jax.ShapeDtypeStruct((N, DX), _f32),
            jax.ShapeDtypeStruct((N, H), _f32),
            jax.ShapeDtypeStruct((1, H), _f32),
        ],
    )(x, w, alf, arf, sel)


def _dstt_body(er_ref, mel_ref, dt_ref):
    er = er_ref[...]
    t = er + mel_ref[...]
    b = jnp.maximum(t, 0.2 * t)
    dt_ref[...] = jnp.concatenate([er, b], axis=1)


def _mk_dstt(er, mel):
    return pl.pallas_call(
        _dstt_body,
        grid=(NB,),
        in_specs=[
            pl.BlockSpec((BN, H), lambda i: (i, 0)),
            pl.BlockSpec((1, H), lambda i: (0, 0)),
        ],
        out_specs=pl.BlockSpec((BN, 2 * H), lambda i: (i, 0)),
        out_shape=jax.ShapeDtypeStruct((N, 2 * H), _f32),
    )(er, mel)


def _dense_mid_body(acc_ref, brow_ref, w_ref, alf_ref, arf_ref,
                    sel_ref, selt_ref,
                    featx_ref, er_ref, mel_ref):
    i = pl.program_id(0)
    accx = acc_ref[0] + acc_ref[1]
    acc = accx[:, :D]
    den = accx[:, D:D + H]
    deninv = 1.0 / (den + 1e-9)
    dexp = jnp.dot(deninv, selt_ref[...], preferred_element_type=_f32)
    h = acc * dexp + brow_ref[...]
    h = jnp.where(h > 0, h, jnp.exp(jnp.minimum(h, 0.0)) - 1.0)
    feat = jnp.dot(h, w_ref[...], preferred_element_type=_f32)
    el = jnp.dot(feat * alf_ref[...], sel_ref[...], preferred_element_type=_f32)
    er = jnp.dot(feat * arf_ref[...], sel_ref[...], preferred_element_type=_f32)
    featx_ref[...] = jnp.concatenate([feat, el, jnp.zeros((BN, H), _f32)], axis=1)
    er_ref[...] = er
    bm = jnp.max(el, axis=0, keepdims=True)

    @pl.when(i == 0)
    def _():
        mel_ref[...] = bm

    @pl.when(i > 0)
    def _():
        mel_ref[...] = jnp.maximum(mel_ref[...], bm)


def _dense_mid(acc, brow, w, alf, arf, sel, selt):
    return pl.pallas_call(
        _dense_mid_body,
        grid=(NB,),
        in_specs=[
            pl.BlockSpec((NC, BN, DX), lambda i: (0, i, 0)),
            pl.BlockSpec((1, D), lambda i: (0, 0)),
            pl.BlockSpec((D, D), lambda i: (0, 0)),
            pl.BlockSpec((1, D), lambda i: (0, 0)),
            pl.BlockSpec((1, D), lambda i: (0, 0)),
            pl.BlockSpec((D, H), lambda i: (0, 0)),
            pl.BlockSpec((H, D), lambda i: (0, 0)),
        ],
        out_specs=[
            pl.BlockSpec((BN, DX), lambda i: (i, 0)),
            pl.BlockSpec((BN, H), lambda i: (i, 0)),
            pl.BlockSpec((1, H), lambda i: (0, 0)),
        ],
        out_shape=[
            jax.ShapeDtypeStruct((N, DX), _f32),
            jax.ShapeDtypeStruct((N, H), _f32),
            jax.ShapeDtypeStruct((1, H), _f32),
        ],
    )(acc, brow, w, alf, arf, sel, selt)


def _dense_fin_body(acc_ref, brow_ref, selt_ref, out_ref):
    accx = acc_ref[0] + acc_ref[1]
    acc = accx[:, :D]
    den = accx[:, D:D + H]
    deninv = 1.0 / (den + 1e-9)
    dexp = jnp.dot(deninv, selt_ref[...], preferred_element_type=_f32)
    h = acc * dexp + brow_ref[...]
    out_ref[...] = jnp.where(h > 0, h, jnp.exp(jnp.minimum(h, 0.0)) - 1.0)


def _dense_fin(acc, brow, selt):
    return pl.pallas_call(
        _dense_fin_body,
        grid=(NB,),
        in_specs=[
            pl.BlockSpec((NC, BN, DX), lambda i: (0, i, 0)),
            pl.BlockSpec((1, D), lambda i: (0, 0)),
            pl.BlockSpec((H, D), lambda i: (0, 0)),
        ],
        out_specs=pl.BlockSpec((BN, D), lambda i: (i, 0)),
        out_shape=jax.ShapeDtypeStruct((N, D), _f32),
    )(acc, brow, selt)


def _sc_edge_body(ed_h, dt_h, featx_h, zx_h,
                  acc_o,
                  ebuf0, dtg0, featg0,
                  ebuf1, dtg1, featg1,
                  acc_sh, semi0, semg0, sems0, semi1, semg1, sems1):
    cid = lax.axis_index("c")
    sid = lax.axis_index("s")
    wid = cid * NS + sid
    row0 = sid * RPT

    pltpu.sync_copy(zx_h.at[pl.ds(row0, RPT)], acc_sh.at[pl.ds(row0, RPT)])
    plsc.subcore_barrier()

    cbase = wid * NCH
    B0 = (ebuf0, dtg0, featg0, semi0, semg0, sems0)
    B1 = (ebuf1, dtg1, featg1, semi1, semg1, sems1)

    def issue_idx(c, b):
        pltpu.async_copy(ed_h.at[cbase + c], b[0], b[3])

    def wait_idx(b):
        pltpu.make_async_copy(ed_h.at[0], b[0], b[3]).wait()

    def issue_gath(b):
        pltpu.async_copy(dt_h.at[b[0].at[1]], b[1], b[4])
        pltpu.async_copy(featx_h.at[b[0].at[0]], b[2], b[4])

    def wait_gath(b):
        pltpu.make_async_copy(dt_h.at[b[0].at[1]], b[1], b[4]).wait()
        pltpu.make_async_copy(featx_h.at[b[0].at[0]], b[2], b[4]).wait()

    def issue_scat(b):
        pltpu.async_copy(b[2], acc_sh.at[b[0].at[1]], b[5], add=True)

    def wait_scat(b):
        pltpu.make_async_copy(b[2], acc_sh.at[b[0].at[1]], b[5]).wait()

    def compute(b):
        ebuf, dtg, featg = b[0], b[1], b[2]

        @plsc.parallel_loop(0, C // 2, 1, unroll=2)
        def pair_body(i):
            it = lax.iota(jnp.int32, 16)
            half = lax.shift_right_logical(it, 3)
            col8 = jnp.bitwise_and(it, 7)
            e0 = 2 * i
            row2 = e0 + half
            el2 = plsc.load_gather(featg, [row2, col8 + D])
            er2 = plsc.load_gather(dtg, [row2, col8])
            b2 = plsc.load_gather(dtg, [row2, col8 + 8])
            ewi = plsc.load_gather(ebuf, [jnp.full((16,), 2, jnp.int32), row2])
            ew2 = plsc.bitcast(ewi, _f32)
            s = el2 + er2
            lr = jnp.maximum(s, 0.2 * s)
            p = jnp.exp(lr - b2)
            q = p * ew2
            plsc.store_scatter(featg, [row2, col8 + D], p)
            for sub in range(2):
                e = e0 + sub
                for hh in range(H):
                    qs = jnp.take(q, jnp.full((16,), sub * H + hh, jnp.int32),
                                  mode="fill")
                    featg[e, pl.ds(hh * DH, DH)] = featg[e, pl.ds(hh * DH, DH)] * qs

    issue_idx(0, B0)
    wait_idx(B0)
    issue_gath(B0)
    NJ = NCH // 2

    def pipe_body(j, carry):
        c1 = 2 * j + 1
        issue_idx(c1, B1)
        wait_gath(B0)
        wait_idx(B1)
        issue_gath(B1)
        compute(B0)
        issue_scat(B0)
        wait_gath(B1)
        compute(B1)
        wait_scat(B0)

        @pl.when(j < NJ - 1)
        def _():
            issue_idx(c1 + 1, B0)

        pltpu.sync_copy(B1[2], acc_sh.at[B1[0].at[1]], add=True)

        @pl.when(j < NJ - 1)
        def _():
            wait_idx(B0)
            issue_gath(B0)

        return carry

    lax.fori_loop(0, NJ, pipe_body, 0)

    if NCH % 2 == 1:
        issue_idx(NCH - 1, B0)
        wait_idx(B0)
        issue_gath(B0)
        wait_gath(B0)
        compute(B0)
        pltpu.sync_copy(B0[2], acc_sh.at[B0[0].at[1]], add=True)

    plsc.subcore_barrier()
    pltpu.sync_copy(acc_sh.at[pl.ds(row0, RPT)], acc_o.at[cid, pl.ds(row0, RPT)])


@functools.cache
def _get_sc_edge_pass():
  return functools.partial(
    pl.kernel,
    out_type=jax.ShapeDtypeStruct((NC, NP, DX), _f32),
    mesh=plsc.VectorSubcoreMesh(core_axis_name="c", subcore_axis_name="s",
                                num_cores=NC, num_subcores=NS),
    compiler_params=pltpu.CompilerParams(use_tc_tiling_on_sc=False, needs_layout_passes=False),
    scratch_types=(
        [pltpu.VMEM((3, C), jnp.int32),
         pltpu.VMEM((C, 2 * H), _f32),
         pltpu.VMEM((C, DX), _f32)] * 2
        + [pltpu.VMEM_SHARED((NP, DX), _f32)]
        + [pltpu.SemaphoreType.DMA] * 6
    ),
  )(_sc_edge_body)


def kernel(in_feat, edge_index, edge_weights, W1, attn_l1, attn_r1, b1,
           W2, attn_l2, attn_r2, b2):
    src = edge_index[0]
    dst = edge_index[1]
    ewbits = lax.bitcast_convert_type(edge_weights, jnp.int32)
    edata = jnp.stack([src.reshape(E // C, C), dst.reshape(E // C, C),
                       ewbits.reshape(E // C, C)], axis=1)
    sel = jnp.repeat(jnp.eye(H, dtype=_f32), DH, axis=0)
    selt = sel.T
    alf1 = attn_l1.reshape(1, D)
    arf1 = attn_r1.reshape(1, D)
    alf2 = attn_l2.reshape(1, D)
    arf2 = attn_r2.reshape(1, D)
    zx = jnp.zeros((NP, DX), _f32)

    featx1, er1, mel1 = _dense_pre(in_feat, W1, alf1, arf1, sel)
    dt1 = _mk_dstt(er1, mel1)
    sc_pass = _get_sc_edge_pass()
    acc1 = sc_pass(edata, dt1, featx1, zx)
    featx2, er2, mel2 = _dense_mid(acc1, b1.reshape(1, D), W2,
                                   alf2, arf2, sel, selt)
    dt2 = _mk_dstt(er2, mel2)
    acc2 = sc_pass(edata, dt2, featx2, zx)
    out = _dense_fin(acc2, b2.reshape(1, D), selt)
    return out


# speedup vs baseline: 1.5633x; 1.4028x over previous
import functools

import jax
import jax.numpy as jnp
from jax import lax
from jax.experimental import pallas as pl
from jax.experimental.pallas import tpu as pltpu
from jax.experimental.pallas import tpu_sc as plsc

N = 10000
E = 320000
D = 128
H = 8
DH = 16

NC, NS = 2, 16
NW = NC * NS
NP = 10000
RPT = NP // NS
EPT = E // NW
C = 80
NCH = EPT // C

BN = 1000
NB = N // BN

_f32 = jnp.float32


def _dense_pre_body(x_ref, w_ref, alf_ref, arf_ref, sel_ref,
                    feat_ref, el_ref, er_ref, mel_ref):
    i = pl.program_id(0)
    feat = jnp.dot(x_ref[...], w_ref[...], preferred_element_type=_f32)
    feat_ref[...] = feat
    el = jnp.dot(feat * alf_ref[...], sel_ref[...], preferred_element_type=_f32)
    er = jnp.dot(feat * arf_ref[...], sel_ref[...], preferred_element_type=_f32)
    el_ref[...] = el
    er_ref[...] = er
    bm = jnp.max(el, axis=0, keepdims=True)

    @pl.when(i == 0)
    def _():
        mel_ref[...] = bm

    @pl.when(i > 0)
    def _():
        mel_ref[...] = jnp.maximum(mel_ref[...], bm)


def _dense_pre(x, w, alf, arf, sel):
    return pl.pallas_call(
        _dense_pre_body,
        grid=(NB,),
        in_specs=[
            pl.BlockSpec((BN, D), lambda i: (i, 0)),
            pl.BlockSpec((D, D), lambda i: (0, 0)),
            pl.BlockSpec((1, D), lambda i: (0, 0)),
            pl.BlockSpec((1, D), lambda i: (0, 0)),
            pl.BlockSpec((D, H), lambda i: (0, 0)),
        ],
        out_specs=[
            pl.BlockSpec((BN, D), lambda i: (i, 0)),
            pl.BlockSpec((BN, H), lambda i: (i, 0)),
            pl.BlockSpec((BN, H), lambda i: (i, 0)),
            pl.BlockSpec((1, H), lambda i: (0, 0)),
        ],
        out_shape=[
            jax.ShapeDtypeStruct((N, D), _f32),
            jax.ShapeDtypeStruct((N, H), _f32),
            jax.ShapeDtypeStruct((N, H), _f32),
            jax.ShapeDtypeStruct((1, H), _f32),
        ],
    )(x, w, alf, arf, sel)


def _dstt_body(er_ref, mel_ref, dt_ref):
    er = er_ref[...]
    t = er + mel_ref[...]
    b = jnp.maximum(t, 0.2 * t)
    dt_ref[...] = jnp.concatenate([er, b], axis=1)


def _mk_dstt(er, mel):
    return pl.pallas_call(
        _dstt_body,
        grid=(NB,),
        in_specs=[
            pl.BlockSpec((BN, H), lambda i: (i, 0)),
            pl.BlockSpec((1, H), lambda i: (0, 0)),
        ],
        out_specs=pl.BlockSpec((BN, 2 * H), lambda i: (i, 0)),
        out_shape=jax.ShapeDtypeStruct((N, 2 * H), _f32),
    )(er, mel)


def _dense_mid_body(acc_ref, den_ref, brow_ref, w_ref, alf_ref, arf_ref,
                    sel_ref, selt_ref,
                    feat_ref, el_ref, er_ref, mel_ref):
    i = pl.program_id(0)
    acc = acc_ref[0] + acc_ref[1]
    den = den_ref[0] + den_ref[1]
    deninv = 1.0 / (den + 1e-9)
    dexp = jnp.dot(deninv, selt_ref[...], preferred_element_type=_f32)
    h = acc * dexp + brow_ref[...]
    h = jnp.where(h > 0, h, jnp.exp(jnp.minimum(h, 0.0)) - 1.0)
    feat = jnp.dot(h, w_ref[...], preferred_element_type=_f32)
    feat_ref[...] = feat
    el = jnp.dot(feat * alf_ref[...], sel_ref[...], preferred_element_type=_f32)
    er = jnp.dot(feat * arf_ref[...], sel_ref[...], preferred_element_type=_f32)
    el_ref[...] = el
    er_ref[...] = er
    bm = jnp.max(el, axis=0, keepdims=True)

    @pl.when(i == 0)
    def _():
        mel_ref[...] = bm

    @pl.when(i > 0)
    def _():
        mel_ref[...] = jnp.maximum(mel_ref[...], bm)


def _dense_mid(acc, den, brow, w, alf, arf, sel, selt):
    return pl.pallas_call(
        _dense_mid_body,
        grid=(NB,),
        in_specs=[
            pl.BlockSpec((NC, BN, D), lambda i: (0, i, 0)),
            pl.BlockSpec((NC, BN, H), lambda i: (0, i, 0)),
            pl.BlockSpec((1, D), lambda i: (0, 0)),
            pl.BlockSpec((D, D), lambda i: (0, 0)),
            pl.BlockSpec((1, D), lambda i: (0, 0)),
            pl.BlockSpec((1, D), lambda i: (0, 0)),
            pl.BlockSpec((D, H), lambda i: (0, 0)),
            pl.BlockSpec((H, D), lambda i: (0, 0)),
        ],
        out_specs=[
            pl.BlockSpec((BN, D), lambda i: (i, 0)),
            pl.BlockSpec((BN, H), lambda i: (i, 0)),
            pl.BlockSpec((BN, H), lambda i: (i, 0)),
            pl.BlockSpec((1, H), lambda i: (0, 0)),
        ],
        out_shape=[
            jax.ShapeDtypeStruct((N, D), _f32),
            jax.ShapeDtypeStruct((N, H), _f32),
            jax.ShapeDtypeStruct((N, H), _f32),
            jax.ShapeDtypeStruct((1, H), _f32),
        ],
    )(acc, den, brow, w, alf, arf, sel, selt)


def _dense_fin_body(acc_ref, den_ref, brow_ref, selt_ref, out_ref):
    acc = acc_ref[0] + acc_ref[1]
    den = den_ref[0] + den_ref[1]
    deninv = 1.0 / (den + 1e-9)
    dexp = jnp.dot(deninv, selt_ref[...], preferred_element_type=_f32)
    h = acc * dexp + brow_ref[...]
    out_ref[...] = jnp.where(h > 0, h, jnp.exp(jnp.minimum(h, 0.0)) - 1.0)


def _dense_fin(acc, den, brow, selt):
    return pl.pallas_call(
        _dense_fin_body,
        grid=(NB,),
        in_specs=[
            pl.BlockSpec((NC, BN, D), lambda i: (0, i, 0)),
            pl.BlockSpec((NC, BN, H), lambda i: (0, i, 0)),
            pl.BlockSpec((1, D), lambda i: (0, 0)),
            pl.BlockSpec((H, D), lambda i: (0, 0)),
        ],
        out_specs=pl.BlockSpec((BN, D), lambda i: (i, 0)),
        out_shape=jax.ShapeDtypeStruct((N, D), _f32),
    )(acc, den, brow, selt)


def _sc_edge_body(src_h, dst_h, ew_h, el_h, dt_h, feat_h, z128_h, z8_h,
                  acc_o, den_o,
                  sidx0, didx0, ewv0, elg0, dtg0, featg0, pch0,
                  sidx1, didx1, ewv1, elg1, dtg1, featg1, pch1,
                  sidx2, didx2, ewv2, elg2, dtg2, featg2, pch2,
                  acc_sh, den_sh,
                  semi0, semg0, sems0, semi1, semg1, sems1, semi2, semg2, sems2):
    cid = lax.axis_index("c")
    sid = lax.axis_index("s")
    wid = cid * NS + sid
    row0 = sid * RPT

    pltpu.sync_copy(z128_h.at[pl.ds(row0, RPT)], acc_sh.at[pl.ds(row0, RPT)])
    pltpu.sync_copy(z8_h.at[pl.ds(row0, RPT)], den_sh.at[pl.ds(row0, RPT)])
    plsc.subcore_barrier()

    ebase = wid * EPT
    B0 = (sidx0, didx0, ewv0, elg0, dtg0, featg0, pch0, semi0, semg0, sems0)
    B1 = (sidx1, didx1, ewv1, elg1, dtg1, featg1, pch1, semi1, semg1, sems1)
    B2 = (sidx2, didx2, ewv2, elg2, dtg2, featg2, pch2, semi2, semg2, sems2)

    def issue_idx(c, b):
        base = ebase + c * C
        pltpu.async_copy(src_h.at[pl.ds(base, C)], b[0], b[7])
        pltpu.async_copy(dst_h.at[pl.ds(base, C)], b[1], b[7])
        pltpu.async_copy(ew_h.at[pl.ds(base, C)], b[2], b[7])

    def wait_idx(b):
        pltpu.make_async_copy(src_h.at[pl.ds(0, C)], b[0], b[7]).wait()
        pltpu.make_async_copy(dst_h.at[pl.ds(0, C)], b[1], b[7]).wait()
        pltpu.make_async_copy(ew_h.at[pl.ds(0, C)], b[2], b[7]).wait()

    def issue_gath(b):
        pltpu.async_copy(el_h.at[b[0]], b[3], b[8])
        pltpu.async_copy(dt_h.at[b[1]], b[4], b[8])
        pltpu.async_copy(feat_h.at[b[0]], b[5], b[8])

    def wait_gath(b):
        pltpu.make_async_copy(el_h.at[b[0]], b[3], b[8]).wait()
        pltpu.make_async_copy(dt_h.at[b[1]], b[4], b[8]).wait()
        pltpu.make_async_copy(feat_h.at[b[0]], b[5], b[8]).wait()

    def issue_scat(b):
        pltpu.async_copy(b[6], den_sh.at[b[1]], b[9], add=True)
        pltpu.async_copy(b[5], acc_sh.at[b[1]], b[9], add=True)

    def wait_scat(b):
        pltpu.make_async_copy(b[6], den_sh.at[b[1]], b[9]).wait()
        pltpu.make_async_copy(b[5], acc_sh.at[b[1]], b[9]).wait()

    def compute(b):
        elg, dtg, featg, pch, ewv = b[3], b[4], b[5], b[6], b[2]

        @plsc.parallel_loop(0, C // 2, 1, unroll=2)
        def pair_body(i):
            it = lax.iota(jnp.int32, 16)
            half = lax.shift_right_logical(it, 3)
            col8 = jnp.bitwise_and(it, 7)
            e0 = 2 * i
            row2 = e0 + half
            el2 = plsc.load_gather(elg, [row2, col8])
            er2 = plsc.load_gather(dtg, [row2, col8])
            b2 = plsc.load_gather(dtg, [row2, col8 + 8])
            ew2 = plsc.load_gather(ewv, [row2])
            s = el2 + er2
            lr = jnp.maximum(s, 0.2 * s)
            p = jnp.exp(lr - b2)
            q = p * ew2
            plsc.store_scatter(pch, [row2, col8], p)
            for sub in range(2):
                e = e0 + sub
                for hh in range(H):
                    qs = jnp.take(q, jnp.full((16,), sub * H + hh, jnp.int32),
                                  mode="fill")
                    featg[e, pl.ds(hh * DH, DH)] = featg[e, pl.ds(hh * DH, DH)] * qs

    NJ = (NCH - 2) // 3
    issue_idx(0, B0)
    wait_idx(B0)
    issue_gath(B0)

    def pipe_body(j, carry):
        a = 3 * j
        issue_idx(a + 1, B1)
        wait_idx(B1)
        issue_gath(B1)

        @pl.when(j > 0)
        def _():
            wait_scat(B2)

        wait_gath(B0)
        compute(B0)
        issue_scat(B0)
        issue_idx(a + 2, B2)
        wait_idx(B2)
        issue_gath(B2)
        wait_gath(B1)
        compute(B1)
        issue_scat(B1)

        @pl.when(j < NJ - 1)
        def _():
            issue_idx(a + 3, B0)

        wait_scat(B0)

        @pl.when(j < NJ - 1)
        def _():
            wait_idx(B0)
            issue_gath(B0)

        wait_gath(B2)
        compute(B2)
        issue_scat(B2)
        wait_scat(B1)
        return carry

    lax.fori_loop(0, NJ, pipe_body, 0)

    for k, b in zip(range(3 * NJ, NCH), (B0, B1)):
        issue_idx(k, b)
        wait_idx(b)
        issue_gath(b)
        wait_gath(b)
        compute(b)
        pltpu.sync_copy(b[6], den_sh.at[b[1]], add=True)
        pltpu.sync_copy(b[5], acc_sh.at[b[1]], add=True)
    wait_scat(B2)

    plsc.subcore_barrier()
    pltpu.sync_copy(acc_sh.at[pl.ds(row0, RPT)], acc_o.at[cid, pl.ds(row0, RPT)])
    pltpu.sync_copy(den_sh.at[pl.ds(row0, RPT)], den_o.at[cid, pl.ds(row0, RPT)])


@functools.cache
def _get_sc_edge_pass():
  return functools.partial(
    pl.kernel,
    out_type=(jax.ShapeDtypeStruct((NC, NP, D), _f32),
              jax.ShapeDtypeStruct((NC, NP, H), _f32)),
    mesh=plsc.VectorSubcoreMesh(core_axis_name="c", subcore_axis_name="s",
                                num_cores=NC, num_subcores=NS),
    compiler_params=pltpu.CompilerParams(use_tc_tiling_on_sc=False, needs_layout_passes=False),
    scratch_types=(
        [pltpu.VMEM((C,), jnp.int32),
         pltpu.VMEM((C,), jnp.int32),
         pltpu.VMEM((C,), _f32),
         pltpu.VMEM((C, H), _f32),
         pltpu.VMEM((C, 2 * H), _f32),
         pltpu.VMEM((C, D), _f32),
         pltpu.VMEM((C, H), _f32)] * 3
        + [pltpu.VMEM_SHARED((NP, D), _f32),
           pltpu.VMEM_SHARED((NP, H), _f32)]
        + [pltpu.SemaphoreType.DMA] * 9
    ),
  )(_sc_edge_body)


def kernel(in_feat, edge_index, edge_weights, W1, attn_l1, attn_r1, b1,
           W2, attn_l2, attn_r2, b2):
    src = edge_index[0]
    dst = edge_index[1]
    sel = jnp.repeat(jnp.eye(H, dtype=_f32), DH, axis=0)
    selt = sel.T
    alf1 = attn_l1.reshape(1, D)
    arf1 = attn_r1.reshape(1, D)
    alf2 = attn_l2.reshape(1, D)
    arf2 = attn_r2.reshape(1, D)
    z128 = jnp.zeros((NP, D), _f32)
    z8 = jnp.zeros((NP, H), _f32)

    feat1, el1, er1, mel1 = _dense_pre(in_feat, W1, alf1, arf1, sel)
    dt1 = _mk_dstt(er1, mel1)
    sc_pass = _get_sc_edge_pass()
    acc1, den1 = sc_pass(src, dst, edge_weights, el1, dt1, feat1, z128, z8)
    feat2, el2, er2, mel2 = _dense_mid(acc1, den1, b1.reshape(1, D), W2,
                                       alf2, arf2, sel, selt)
    dt2 = _mk_dstt(er2, mel2)
    acc2, den2 = sc_pass(src, dst, edge_weights, el2, dt2, feat2, z128, z8)
    out = _dense_fin(acc2, den2, b2.reshape(1, D), selt)
    return out
